# Initial kernel scaffold; baseline (speedup 1.0000x reference)
import jax
import jax.numpy as jnp
from jax.experimental import pallas as pl


def kernel(inputs, lengths, emb, Wih_l0f, Whh_l0f, bih_l0f, bhh_l0f, Wih_l0b, Whh_l0b, bih_l0b, bhh_l0b, Wih_l1f, Whh_l1f, bih_l1f, bhh_l1f, Wih_l1b, Whh_l1b, bih_l1b, bhh_l1b, fc_w, fc_b, cls_w, cls_b):
    raise NotImplementedError("write your pallas kernel here")


# speedup vs baseline: 3.0625x; 3.0625x over previous
import functools

import jax
import jax.numpy as jnp
from jax.experimental import pallas as pl
from jax.experimental.pallas import tpu as pltpu
from jax.experimental.pallas import tpu_sc as plsc

B, T, VOCAB, D_EMB, H, L_OUT, TAGS = 64, 128, 100000, 256, 512, 256, 50
G = 4 * H
N = T * B
PREC = jax.lax.Precision.HIGHEST

_GATHER_WINDOW = 128


def _sc_gather(emb, idx):
    mesh = plsc.VectorSubcoreMesh(core_axis_name="core", subcore_axis_name="subcore")

    @pl.kernel(out_type=jax.ShapeDtypeStruct((N, D_EMB), emb.dtype), mesh=mesh)
    def gather_kernel(emb_hbm, i_hbm, o_hbm):
        def body(i_vmem, o_vmem):
            pltpu.sync_copy(emb_hbm.at[i_vmem.at[0]], o_vmem)

        pltpu.emit_pipeline(
            body,
            grid=(N // _GATHER_WINDOW,),
            in_specs=[pl.BlockSpec((1, _GATHER_WINDOW), index_map=lambda i: (0, i))],
            out_specs=[pl.BlockSpec((_GATHER_WINDOW, D_EMB), index_map=lambda i: (i, 0))],
            core_axis_name=("core", "subcore"),
            dimension_semantics=(pltpu.PARALLEL,),
        )(i_hbm, o_hbm)

    return gather_kernel(emb, idx.reshape(1, N))


def _dot(a, b):
    return jax.lax.dot_general(a, b, (((1,), (0,)), ((), ())),
                               precision=PREC, preferred_element_type=jnp.float32)


def _proj1_body(x_ref, w_ref, b_ref, o_ref):
    o_ref[...] = _dot(x_ref[...], w_ref[...]) + b_ref[...]


def _proj1(x, w, b, bm):
    m, k = x.shape
    n = w.shape[1]
    return pl.pallas_call(
        _proj1_body,
        grid=(m // bm,),
        in_specs=[
            pl.BlockSpec((bm, k), lambda i: (i, 0)),
            pl.BlockSpec((k, n), lambda i: (0, 0)),
            pl.BlockSpec((1, n), lambda i: (0, 0)),
        ],
        out_specs=pl.BlockSpec((bm, n), lambda i: (i, 0)),
        out_shape=jax.ShapeDtypeStruct((m, n), jnp.float32),
    )(x, w, b.reshape(1, n))


def _proj2_body(xa_ref, xb_ref, wa_ref, wb_ref, b_ref, o_ref):
    o_ref[...] = (_dot(xa_ref[...], wa_ref[...]) + _dot(xb_ref[...], wb_ref[...])
                  + b_ref[...])


def _proj2(xa, xb, wa, wb, b, bm):
    m, k = xa.shape
    n = wa.shape[1]
    return pl.pallas_call(
        _proj2_body,
        grid=(m // bm,),
        in_specs=[
            pl.BlockSpec((bm, k), lambda i: (i, 0)),
            pl.BlockSpec((bm, k), lambda i: (i, 0)),
            pl.BlockSpec((k, n), lambda i: (0, 0)),
            pl.BlockSpec((k, n), lambda i: (0, 0)),
            pl.BlockSpec((1, n), lambda i: (0, 0)),
        ],
        out_specs=pl.BlockSpec((bm, n), lambda i: (i, 0)),
        out_shape=jax.ShapeDtypeStruct((m, n), jnp.float32),
    )(xa, xb, wa, wb, b.reshape(1, n))


def _rec_body(len_ref, xf_ref, xb_ref, wf_ref, wb_ref, yf_ref, yb_ref,
              hf, cf, hb, cb):
    i = pl.program_id(0)

    @pl.when(i == 0)
    def _():
        zero = jnp.zeros((B, H), jnp.float32)
        hf[...] = zero
        cf[...] = zero
        hb[...] = zero
        cb[...] = zero

    lens = len_ref[...]

    def step(x_ref, w_ref, h_ref, c_ref, y_ref, t):
        g = x_ref[0] + _dot(h_ref[...], w_ref[...])
        gi = jax.nn.sigmoid(g[:, :H])
        gf = jax.nn.sigmoid(g[:, H:2 * H])
        gg = jnp.tanh(g[:, 2 * H:3 * H])
        go = jax.nn.sigmoid(g[:, 3 * H:])
        c_new = gf * c_ref[...] + gi * gg
        h_new = go * jnp.tanh(c_new)
        m = (lens > t.astype(jnp.float32)).astype(jnp.float32)
        y_ref[0] = h_new * m
        h_ref[...] = m * h_new + (1.0 - m) * h_ref[...]
        c_ref[...] = m * c_new + (1.0 - m) * c_ref[...]

    step(xf_ref, wf_ref, hf, cf, yf_ref, i)
    step(xb_ref, wb_ref, hb, cb, yb_ref, T - 1 - i)


def _birecur(xp, whf_t, whb_t, len_col):
    yf, yb = pl.pallas_call(
        _rec_body,
        grid=(T,),
        in_specs=[
            pl.BlockSpec((B, 1), lambda i: (0, 0)),
            pl.BlockSpec((1, B, G), lambda i: (i, 0, 0)),
            pl.BlockSpec((1, B, G), lambda i: (T - 1 - i, 0, 1)),
            pl.BlockSpec((H, G), lambda i: (0, 0)),
            pl.BlockSpec((H, G), lambda i: (0, 0)),
        ],
        out_specs=[
            pl.BlockSpec((1, B, H), lambda i: (i, 0, 0)),
            pl.BlockSpec((1, B, H), lambda i: (T - 1 - i, 0, 0)),
        ],
        out_shape=[
            jax.ShapeDtypeStruct((T, B, H), jnp.float32),
            jax.ShapeDtypeStruct((T, B, H), jnp.float32),
        ],
        scratch_shapes=[pltpu.VMEM((B, H), jnp.float32) for _ in range(4)],
        compiler_params=pltpu.CompilerParams(
            dimension_semantics=("arbitrary",)),
    )(len_col, xp, xp, whf_t, whb_t)
    return yf, yb


def _head_body(ya_ref, yb_ref, wa_ref, wb_ref, fcb_ref, cls_ref, clsb_ref, o_ref):
    h = _dot(ya_ref[...], wa_ref[...]) + _dot(yb_ref[...], wb_ref[...]) + fcb_ref[...]
    h = jnp.where(h > 0, h, 0.01 * (jnp.exp(jnp.minimum(h, 0.0)) - 1.0))
    o_ref[...] = _dot(h, cls_ref[...]) + clsb_ref[...]


def _head(ya, yb, wa, wb, fcb, cls_t, clsb, bm):
    m = ya.shape[0]
    n = cls_t.shape[1]
    return pl.pallas_call(
        _head_body,
        grid=(m // bm,),
        in_specs=[
            pl.BlockSpec((bm, H), lambda i: (i, 0)),
            pl.BlockSpec((bm, H), lambda i: (i, 0)),
            pl.BlockSpec((H, L_OUT), lambda i: (0, 0)),
            pl.BlockSpec((H, L_OUT), lambda i: (0, 0)),
            pl.BlockSpec((1, L_OUT), lambda i: (0, 0)),
            pl.BlockSpec((L_OUT, n), lambda i: (0, 0)),
            pl.BlockSpec((1, n), lambda i: (0, 0)),
        ],
        out_specs=pl.BlockSpec((bm, n), lambda i: (i, 0)),
        out_shape=jax.ShapeDtypeStruct((m, n), jnp.float32),
    )(ya, yb, wa, wb, fcb.reshape(1, L_OUT), cls_t, clsb.reshape(1, n))


def kernel(inputs, lengths, emb, Wih_l0f, Whh_l0f, bih_l0f, bhh_l0f,
           Wih_l0b, Whh_l0b, bih_l0b, bhh_l0b,
           Wih_l1f, Whh_l1f, bih_l1f, bhh_l1f,
           Wih_l1b, Whh_l1b, bih_l1b, bhh_l1b,
           fc_w, fc_b, cls_w, cls_b):
    f32 = jnp.float32
    idx = inputs.T.reshape(N).astype(jnp.int32)
    len_col = lengths.astype(f32).reshape(B, 1)

    w0 = jnp.concatenate([Wih_l0f.T, Wih_l0b.T], axis=1)
    b0 = jnp.concatenate([bih_l0f + bhh_l0f, bih_l0b + bhh_l0b])
    w1_top = jnp.concatenate([Wih_l1f[:, :H].T, Wih_l1b[:, :H].T], axis=1)
    w1_bot = jnp.concatenate([Wih_l1f[:, H:].T, Wih_l1b[:, H:].T], axis=1)
    b1 = jnp.concatenate([bih_l1f + bhh_l1f, bih_l1b + bhh_l1b])
    fc_a = fc_w[:, :H].T
    fc_b2 = fc_w[:, H:].T
    n_pad = 128
    cls_t = jnp.zeros((L_OUT, n_pad), f32).at[:, :TAGS].set(cls_w.T)
    clsb_pad = jnp.zeros((n_pad,), f32).at[:TAGS].set(cls_b)

    x0 = _sc_gather(emb, idx)

    xp0 = _proj1(x0, w0, b0, bm=512).reshape(T, B, 2 * G)
    yf0, yb0 = _birecur(xp0, Whh_l0f.T, Whh_l0b.T, len_col)

    xp1 = _proj2(yf0.reshape(N, H), yb0.reshape(N, H), w1_top, w1_bot,
                 b1, bm=512).reshape(T, B, 2 * G)
    yf1, yb1 = _birecur(xp1, Whh_l1f.T, Whh_l1b.T, len_col)

    out = _head(yf1.reshape(N, H), yb1.reshape(N, H), fc_a, fc_b2,
                fc_b, cls_t, clsb_pad, bm=1024)
    return out.reshape(T, B, n_pad).transpose(1, 0, 2)[:, :, :TAGS]


# speedup vs baseline: 10.7597x; 3.5133x over previous
import functools

import jax
import jax.numpy as jnp
from jax.experimental import pallas as pl
from jax.experimental.pallas import tpu as pltpu
from jax.experimental.pallas import tpu_sc as plsc

B, T, VOCAB, D_EMB, H, L_OUT, TAGS = 64, 128, 100000, 256, 512, 256, 50
G = 4 * H
N = T * B
PREC = jax.lax.Precision.DEFAULT

_GATHER_WINDOW = 128


def _sc_gather(emb, idx):
    mesh = plsc.VectorSubcoreMesh(core_axis_name="core", subcore_axis_name="subcore")

    @pl.kernel(out_type=jax.ShapeDtypeStruct((N, D_EMB), emb.dtype), mesh=mesh)
    def gather_kernel(emb_hbm, i_hbm, o_hbm):
        def body(i_vmem, o_vmem):
            pltpu.sync_copy(emb_hbm.at[i_vmem.at[0]], o_vmem)

        pltpu.emit_pipeline(
            body,
            grid=(N // _GATHER_WINDOW,),
            in_specs=[pl.BlockSpec((1, _GATHER_WINDOW), index_map=lambda i: (0, i))],
            out_specs=[pl.BlockSpec((_GATHER_WINDOW, D_EMB), index_map=lambda i: (i, 0))],
            core_axis_name=("core", "subcore"),
            dimension_semantics=(pltpu.PARALLEL,),
        )(i_hbm, o_hbm)

    return gather_kernel(emb, idx.reshape(1, N))


def _dot(a, b):
    return jax.lax.dot_general(a, b, (((1,), (0,)), ((), ())),
                               precision=PREC, preferred_element_type=jnp.float32)


def _proj1_body(x_ref, w_ref, b_ref, o_ref):
    o_ref[...] = _dot(x_ref[...], w_ref[...]) + b_ref[...]


def _proj1(x, w, b, bm):
    m, k = x.shape
    n = w.shape[1]
    return pl.pallas_call(
        _proj1_body,
        grid=(m // bm,),
        in_specs=[
            pl.BlockSpec((bm, k), lambda i: (i, 0)),
            pl.BlockSpec((k, n), lambda i: (0, 0)),
            pl.BlockSpec((1, n), lambda i: (0, 0)),
        ],
        out_specs=pl.BlockSpec((bm, n), lambda i: (i, 0)),
        out_shape=jax.ShapeDtypeStruct((m, n), jnp.float32),
    )(x, w, b.reshape(1, n))


def _proj2_body(xa_ref, xb_ref, wa_ref, wb_ref, b_ref, o_ref):
    o_ref[...] = (_dot(xa_ref[...], wa_ref[...]) + _dot(xb_ref[...], wb_ref[...])
                  + b_ref[...])


def _proj2(xa, xb, wa, wb, b, bm):
    m, k = xa.shape
    n = wa.shape[1]
    return pl.pallas_call(
        _proj2_body,
        grid=(m // bm,),
        in_specs=[
            pl.BlockSpec((bm, k), lambda i: (i, 0)),
            pl.BlockSpec((bm, k), lambda i: (i, 0)),
            pl.BlockSpec((k, n), lambda i: (0, 0)),
            pl.BlockSpec((k, n), lambda i: (0, 0)),
            pl.BlockSpec((1, n), lambda i: (0, 0)),
        ],
        out_specs=pl.BlockSpec((bm, n), lambda i: (i, 0)),
        out_shape=jax.ShapeDtypeStruct((m, n), jnp.float32),
    )(xa, xb, wa, wb, b.reshape(1, n))


def _rec_body(len_ref, xf_ref, xb_ref, wf_ref, wb_ref, yf_ref, yb_ref,
              hf, cf, hb, cb):
    i = pl.program_id(0)

    @pl.when(i == 0)
    def _():
        zero = jnp.zeros((B, H), jnp.float32)
        hf[...] = zero
        cf[...] = zero
        hb[...] = zero
        cb[...] = zero

    lens = len_ref[...]

    def step(x_ref, w_ref, h_ref, c_ref, y_ref, t):
        g = x_ref[0] + _dot(h_ref[...], w_ref[...])
        gi = jax.nn.sigmoid(g[:, :H])
        gf = jax.nn.sigmoid(g[:, H:2 * H])
        gg = jnp.tanh(g[:, 2 * H:3 * H])
        go = jax.nn.sigmoid(g[:, 3 * H:])
        c_new = gf * c_ref[...] + gi * gg
        h_new = go * jnp.tanh(c_new)
        m = (lens > t.astype(jnp.float32)).astype(jnp.float32)
        y_ref[0] = h_new * m
        h_ref[...] = m * h_new + (1.0 - m) * h_ref[...]
        c_ref[...] = m * c_new + (1.0 - m) * c_ref[...]

    step(xf_ref, wf_ref, hf, cf, yf_ref, i)
    step(xb_ref, wb_ref, hb, cb, yb_ref, T - 1 - i)


def _birecur(xp, whf_t, whb_t, len_col):
    yf, yb = pl.pallas_call(
        _rec_body,
        grid=(T,),
        in_specs=[
            pl.BlockSpec((B, 1), lambda i: (0, 0)),
            pl.BlockSpec((1, B, G), lambda i: (i, 0, 0)),
            pl.BlockSpec((1, B, G), lambda i: (T - 1 - i, 0, 1)),
            pl.BlockSpec((H, G), lambda i: (0, 0)),
            pl.BlockSpec((H, G), lambda i: (0, 0)),
        ],
        out_specs=[
            pl.BlockSpec((1, B, H), lambda i: (i, 0, 0)),
            pl.BlockSpec((1, B, H), lambda i: (T - 1 - i, 0, 0)),
        ],
        out_shape=[
            jax.ShapeDtypeStruct((T, B, H), jnp.float32),
            jax.ShapeDtypeStruct((T, B, H), jnp.float32),
        ],
        scratch_shapes=[pltpu.VMEM((B, H), jnp.float32) for _ in range(4)],
        compiler_params=pltpu.CompilerParams(
            dimension_semantics=("arbitrary",)),
    )(len_col, xp, xp, whf_t, whb_t)
    return yf, yb


def _head_body(ya_ref, yb_ref, wa_ref, wb_ref, fcb_ref, cls_ref, clsb_ref, o_ref):
    h = _dot(ya_ref[...], wa_ref[...]) + _dot(yb_ref[...], wb_ref[...]) + fcb_ref[...]
    h = jnp.where(h > 0, h, 0.01 * (jnp.exp(jnp.minimum(h, 0.0)) - 1.0))
    o_ref[...] = _dot(h, cls_ref[...]) + clsb_ref[...]


def _head(ya, yb, wa, wb, fcb, cls_t, clsb, bm):
    m = ya.shape[0]
    n = cls_t.shape[1]
    return pl.pallas_call(
        _head_body,
        grid=(m // bm,),
        in_specs=[
            pl.BlockSpec((bm, H), lambda i: (i, 0)),
            pl.BlockSpec((bm, H), lambda i: (i, 0)),
            pl.BlockSpec((H, L_OUT), lambda i: (0, 0)),
            pl.BlockSpec((H, L_OUT), lambda i: (0, 0)),
            pl.BlockSpec((1, L_OUT), lambda i: (0, 0)),
            pl.BlockSpec((L_OUT, n), lambda i: (0, 0)),
            pl.BlockSpec((1, n), lambda i: (0, 0)),
        ],
        out_specs=pl.BlockSpec((bm, n), lambda i: (i, 0)),
        out_shape=jax.ShapeDtypeStruct((m, n), jnp.float32),
    )(ya, yb, wa, wb, fcb.reshape(1, L_OUT), cls_t, clsb.reshape(1, n))


def kernel(inputs, lengths, emb, Wih_l0f, Whh_l0f, bih_l0f, bhh_l0f,
           Wih_l0b, Whh_l0b, bih_l0b, bhh_l0b,
           Wih_l1f, Whh_l1f, bih_l1f, bhh_l1f,
           Wih_l1b, Whh_l1b, bih_l1b, bhh_l1b,
           fc_w, fc_b, cls_w, cls_b):
    f32 = jnp.float32
    idx = inputs.T.reshape(N).astype(jnp.int32)
    len_col = lengths.astype(f32).reshape(B, 1)

    w0 = jnp.concatenate([Wih_l0f.T, Wih_l0b.T], axis=1)
    b0 = jnp.concatenate([bih_l0f + bhh_l0f, bih_l0b + bhh_l0b])
    w1_top = jnp.concatenate([Wih_l1f[:, :H].T, Wih_l1b[:, :H].T], axis=1)
    w1_bot = jnp.concatenate([Wih_l1f[:, H:].T, Wih_l1b[:, H:].T], axis=1)
    b1 = jnp.concatenate([bih_l1f + bhh_l1f, bih_l1b + bhh_l1b])
    fc_a = fc_w[:, :H].T
    fc_b2 = fc_w[:, H:].T
    n_pad = 128
    cls_t = jnp.zeros((L_OUT, n_pad), f32).at[:, :TAGS].set(cls_w.T)
    clsb_pad = jnp.zeros((n_pad,), f32).at[:TAGS].set(cls_b)

    x0 = _sc_gather(emb, idx)

    xp0 = _proj1(x0, w0, b0, bm=512).reshape(T, B, 2 * G)
    yf0, yb0 = _birecur(xp0, Whh_l0f.T, Whh_l0b.T, len_col)

    xp1 = _proj2(yf0.reshape(N, H), yb0.reshape(N, H), w1_top, w1_bot,
                 b1, bm=512).reshape(T, B, 2 * G)
    yf1, yb1 = _birecur(xp1, Whh_l1f.T, Whh_l1b.T, len_col)

    out = _head(yf1.reshape(N, H), yb1.reshape(N, H), fc_a, fc_b2,
                fc_b, cls_t, clsb_pad, bm=1024)
    return out.reshape(T, B, n_pad).transpose(1, 0, 2)[:, :, :TAGS]


# speedup vs baseline: 13.8697x; 1.2890x over previous
import functools

import jax
import jax.numpy as jnp
from jax.experimental import pallas as pl
from jax.experimental.pallas import tpu as pltpu
from jax.experimental.pallas import tpu_sc as plsc

B, T, VOCAB, D_EMB, H, L_OUT, TAGS = 64, 128, 100000, 256, 512, 256, 50
G = 4 * H
N = T * B
PREC = jax.lax.Precision.DEFAULT

_GATHER_WINDOW = 128


def _sc_gather(emb, idx):
    mesh = plsc.VectorSubcoreMesh(core_axis_name="core", subcore_axis_name="subcore")

    @pl.kernel(out_type=jax.ShapeDtypeStruct((N, D_EMB), emb.dtype), mesh=mesh)
    def gather_kernel(emb_hbm, i_hbm, o_hbm):
        def body(i_vmem, o_vmem):
            pltpu.sync_copy(emb_hbm.at[i_vmem.at[0]], o_vmem)

        pltpu.emit_pipeline(
            body,
            grid=(N // _GATHER_WINDOW,),
            in_specs=[pl.BlockSpec((1, _GATHER_WINDOW), index_map=lambda i: (0, i))],
            out_specs=[pl.BlockSpec((_GATHER_WINDOW, D_EMB), index_map=lambda i: (i, 0))],
            core_axis_name=("core", "subcore"),
            dimension_semantics=(pltpu.PARALLEL,),
        )(i_hbm, o_hbm)

    return gather_kernel(emb, idx.reshape(1, N))


def _dot(a, b):
    return jax.lax.dot_general(a, b, (((1,), (0,)), ((), ())),
                               precision=PREC, preferred_element_type=jnp.float32)


def _proj1_body(x_ref, w_ref, b_ref, o_ref):
    o_ref[...] = (_dot(x_ref[...], w_ref[...]) + b_ref[...]).astype(o_ref.dtype)


def _proj1(x, w, b, bm):
    m, k = x.shape
    n = w.shape[1]
    return pl.pallas_call(
        _proj1_body,
        grid=(m // bm,),
        in_specs=[
            pl.BlockSpec((bm, k), lambda i: (i, 0)),
            pl.BlockSpec((k, n), lambda i: (0, 0)),
            pl.BlockSpec((1, n), lambda i: (0, 0)),
        ],
        out_specs=pl.BlockSpec((bm, n), lambda i: (i, 0)),
        out_shape=jax.ShapeDtypeStruct((m, n), jnp.bfloat16),
    )(x, w, b.reshape(1, n))


def _proj2_body(xa_ref, xb_ref, wa_ref, wb_ref, b_ref, o_ref):
    o_ref[...] = (_dot(xa_ref[...], wa_ref[...]) + _dot(xb_ref[...], wb_ref[...])
                  + b_ref[...]).astype(o_ref.dtype)


def _proj2(xa, xb, wa, wb, b, bm):
    m, k = xa.shape
    n = wa.shape[1]
    return pl.pallas_call(
        _proj2_body,
        grid=(m // bm,),
        in_specs=[
            pl.BlockSpec((bm, k), lambda i: (i, 0)),
            pl.BlockSpec((bm, k), lambda i: (i, 0)),
            pl.BlockSpec((k, n), lambda i: (0, 0)),
            pl.BlockSpec((k, n), lambda i: (0, 0)),
            pl.BlockSpec((1, n), lambda i: (0, 0)),
        ],
        out_specs=pl.BlockSpec((bm, n), lambda i: (i, 0)),
        out_shape=jax.ShapeDtypeStruct((m, n), jnp.bfloat16),
    )(xa, xb, wa, wb, b.reshape(1, n))


def _rec_body(len_ref, xf_ref, xb_ref, wf_ref, wb_ref, yf_ref, yb_ref,
              hf, cf, hb, cb):
    i = pl.program_id(0)

    @pl.when(i == 0)
    def _():
        zero = jnp.zeros((B, H), jnp.float32)
        hf[...] = zero
        cf[...] = zero
        hb[...] = zero
        cb[...] = zero

    lens = len_ref[...]

    def gates(g, c):
        gi = jax.nn.sigmoid(g[:, :H])
        gf = jax.nn.sigmoid(g[:, H:2 * H])
        gg = jnp.tanh(g[:, 2 * H:3 * H])
        go = jax.nn.sigmoid(g[:, 3 * H:])
        c_new = gf * c + gi * gg
        h_new = go * jnp.tanh(c_new)
        return h_new, c_new

    def pair(kf, kb, tf, tb):
        g_f = xf_ref[kf].astype(jnp.float32) + _dot(hf[...].astype(jnp.bfloat16),
                                                    wf_ref[...])
        g_b = xb_ref[kb].astype(jnp.float32) + _dot(hb[...].astype(jnp.bfloat16),
                                                    wb_ref[...])
        hn_f, cn_f = gates(g_f, cf[...])
        hn_b, cn_b = gates(g_b, cb[...])
        m_f = (lens > jnp.float32(tf)).astype(jnp.float32)
        m_b = (lens > jnp.float32(tb)).astype(jnp.float32)
        yf_ref[kf] = (hn_f * m_f).astype(yf_ref.dtype)
        yb_ref[kb] = (hn_b * m_b).astype(yb_ref.dtype)
        hf[...] = m_f * hn_f + (1.0 - m_f) * hf[...]
        cf[...] = m_f * cn_f + (1.0 - m_f) * cf[...]
        hb[...] = m_b * hn_b + (1.0 - m_b) * hb[...]
        cb[...] = m_b * cn_b + (1.0 - m_b) * cb[...]

    t0 = 2 * i
    pair(0, 1, t0, T - 1 - t0)
    pair(1, 0, t0 + 1, T - 2 - t0)


def _birecur(xp, whf_t, whb_t, len_col):
    yf, yb = pl.pallas_call(
        _rec_body,
        grid=(T // 2,),
        in_specs=[
            pl.BlockSpec((B, 1), lambda i: (0, 0)),
            pl.BlockSpec((2, B, G), lambda i: (i, 0, 0)),
            pl.BlockSpec((2, B, G), lambda i: (T // 2 - 1 - i, 0, 1)),
            pl.BlockSpec((H, G), lambda i: (0, 0)),
            pl.BlockSpec((H, G), lambda i: (0, 0)),
        ],
        out_specs=[
            pl.BlockSpec((2, B, H), lambda i: (i, 0, 0)),
            pl.BlockSpec((2, B, H), lambda i: (T // 2 - 1 - i, 0, 0)),
        ],
        out_shape=[
            jax.ShapeDtypeStruct((T, B, H), jnp.bfloat16),
            jax.ShapeDtypeStruct((T, B, H), jnp.bfloat16),
        ],
        scratch_shapes=[pltpu.VMEM((B, H), jnp.float32) for _ in range(4)],
        compiler_params=pltpu.CompilerParams(
            dimension_semantics=("arbitrary",)),
    )(len_col, xp, xp, whf_t, whb_t)
    return yf, yb


def _head_body(ya_ref, yb_ref, wa_ref, wb_ref, fcb_ref, cls_ref, clsb_ref, o_ref):
    h = _dot(ya_ref[...], wa_ref[...]) + _dot(yb_ref[...], wb_ref[...]) + fcb_ref[...]
    h = jnp.where(h > 0, h, 0.01 * (jnp.exp(jnp.minimum(h, 0.0)) - 1.0))
    o_ref[...] = _dot(h, cls_ref[...]) + clsb_ref[...]


def _head(ya, yb, wa, wb, fcb, cls_t, clsb, bm):
    m = ya.shape[0]
    n = cls_t.shape[1]
    return pl.pallas_call(
        _head_body,
        grid=(m // bm,),
        in_specs=[
            pl.BlockSpec((bm, H), lambda i: (i, 0)),
            pl.BlockSpec((bm, H), lambda i: (i, 0)),
            pl.BlockSpec((H, L_OUT), lambda i: (0, 0)),
            pl.BlockSpec((H, L_OUT), lambda i: (0, 0)),
            pl.BlockSpec((1, L_OUT), lambda i: (0, 0)),
            pl.BlockSpec((L_OUT, n), lambda i: (0, 0)),
            pl.BlockSpec((1, n), lambda i: (0, 0)),
        ],
        out_specs=pl.BlockSpec((bm, n), lambda i: (i, 0)),
        out_shape=jax.ShapeDtypeStruct((m, n), jnp.float32),
    )(ya, yb, wa, wb, fcb.reshape(1, L_OUT), cls_t, clsb.reshape(1, n))


def kernel(inputs, lengths, emb, Wih_l0f, Whh_l0f, bih_l0f, bhh_l0f,
           Wih_l0b, Whh_l0b, bih_l0b, bhh_l0b,
           Wih_l1f, Whh_l1f, bih_l1f, bhh_l1f,
           Wih_l1b, Whh_l1b, bih_l1b, bhh_l1b,
           fc_w, fc_b, cls_w, cls_b):
    f32 = jnp.float32
    bf16 = jnp.bfloat16
    idx = inputs.T.reshape(N).astype(jnp.int32)
    len_col = lengths.astype(f32).reshape(B, 1)

    w0 = jnp.concatenate([Wih_l0f.T, Wih_l0b.T], axis=1)
    b0 = jnp.concatenate([bih_l0f + bhh_l0f, bih_l0b + bhh_l0b])
    w1_top = jnp.concatenate([Wih_l1f[:, :H].T, Wih_l1b[:, :H].T], axis=1)
    w1_bot = jnp.concatenate([Wih_l1f[:, H:].T, Wih_l1b[:, H:].T], axis=1)
    b1 = jnp.concatenate([bih_l1f + bhh_l1f, bih_l1b + bhh_l1b])
    fc_a = fc_w[:, :H].T
    fc_b2 = fc_w[:, H:].T
    n_pad = 128
    cls_t = jnp.zeros((L_OUT, n_pad), f32).at[:, :TAGS].set(cls_w.T)
    clsb_pad = jnp.zeros((n_pad,), f32).at[:TAGS].set(cls_b)

    x0 = _sc_gather(emb, idx)

    xp0 = _proj1(x0, w0, b0, bm=512).reshape(T, B, 2 * G)
    yf0, yb0 = _birecur(xp0, Whh_l0f.T.astype(bf16), Whh_l0b.T.astype(bf16),
                        len_col)

    xp1 = _proj2(yf0.reshape(N, H), yb0.reshape(N, H), w1_top.astype(bf16),
                 w1_bot.astype(bf16), b1, bm=512).reshape(T, B, 2 * G)
    yf1, yb1 = _birecur(xp1, Whh_l1f.T.astype(bf16), Whh_l1b.T.astype(bf16),
                        len_col)

    out = _head(yf1.reshape(N, H), yb1.reshape(N, H), fc_a.astype(bf16),
                fc_b2.astype(bf16), fc_b, cls_t, clsb_pad, bm=1024)
    return out.reshape(T, B, n_pad).transpose(1, 0, 2)[:, :, :TAGS]


# speedup vs baseline: 14.9013x; 1.0744x over previous
import functools

import jax
import jax.numpy as jnp
from jax.experimental import pallas as pl
from jax.experimental.pallas import tpu as pltpu
from jax.experimental.pallas import tpu_sc as plsc

B, T, VOCAB, D_EMB, H, L_OUT, TAGS = 64, 128, 100000, 256, 512, 256, 50
G = 4 * H
N = T * B
UNROLL = 4
NB = T // UNROLL

_GATHER_WINDOW = 128


def _sc_gather(emb, idx):
    mesh = plsc.VectorSubcoreMesh(core_axis_name="core", subcore_axis_name="subcore")

    @pl.kernel(out_type=jax.ShapeDtypeStruct((N, D_EMB), emb.dtype), mesh=mesh)
    def gather_kernel(emb_hbm, i_hbm, o_hbm):
        def body(i_vmem, o_vmem):
            pltpu.sync_copy(emb_hbm.at[i_vmem.at[0]], o_vmem)

        pltpu.emit_pipeline(
            body,
            grid=(N // _GATHER_WINDOW,),
            in_specs=[pl.BlockSpec((1, _GATHER_WINDOW), index_map=lambda i: (0, i))],
            out_specs=[pl.BlockSpec((_GATHER_WINDOW, D_EMB), index_map=lambda i: (i, 0))],
            core_axis_name=("core", "subcore"),
            dimension_semantics=(pltpu.PARALLEL,),
        )(i_hbm, o_hbm)

    return gather_kernel(emb, idx.reshape(1, N))


def _dot(a, b):
    return jax.lax.dot_general(a, b, (((1,), (0,)), ((), ())),
                               preferred_element_type=jnp.float32)


def _dot_t(a, b):
    return jax.lax.dot_general(a, b, (((1,), (1,)), ((), ())),
                               preferred_element_type=jnp.float32)


def _recurrence_block(i, len_ref, sxf, sxb, yf_ref, yb_ref, hf, cf, hb, cb,
                      whf_ref, whb_ref):
    @pl.when(i == 1)
    def _():
        zero = jnp.zeros((B, H), jnp.float32)
        hf[...] = zero
        cf[...] = zero
        hb[...] = zero
        cb[...] = zero

    lens = len_ref[...]
    buf = (i - 1) % 2

    def gates(g, c):
        gi = jax.nn.sigmoid(g[:, :H])
        gf = jax.nn.sigmoid(g[:, H:2 * H])
        gg = jnp.tanh(g[:, 2 * H:3 * H])
        go = jax.nn.sigmoid(g[:, 3 * H:])
        c_new = gf * c + gi * gg
        h_new = go * jnp.tanh(c_new)
        return h_new, c_new

    t0 = (i - 1) * UNROLL
    for k in range(UNROLL):
        tf = t0 + k
        tb = T - 1 - tf
        g_f = sxf[buf, pl.ds(B * k, B), :] + _dot(hf[...].astype(jnp.bfloat16),
                                                  whf_ref[...])
        g_b = (sxb[buf, pl.ds(B * (UNROLL - 1 - k), B), :]
               + _dot(hb[...].astype(jnp.bfloat16), whb_ref[...]))
        hn_f, cn_f = gates(g_f, cf[...])
        hn_b, cn_b = gates(g_b, cb[...])
        m_f = (lens > jnp.float32(tf)).astype(jnp.float32)
        m_b = (lens > jnp.float32(tb)).astype(jnp.float32)
        yf_ref[k] = (hn_f * m_f).astype(yf_ref.dtype)
        yb_ref[UNROLL - 1 - k] = (hn_b * m_b).astype(yb_ref.dtype)
        hf[...] = m_f * hn_f + (1.0 - m_f) * hf[...]
        cf[...] = m_f * cn_f + (1.0 - m_f) * cf[...]
        hb[...] = m_b * hn_b + (1.0 - m_b) * hb[...]
        cb[...] = m_b * cn_b + (1.0 - m_b) * cb[...]


def _layer0_body(len_ref, pf_ref, pb_ref, wif_ref, wib_ref, whf_ref, whb_ref,
                 b_ref, yf_ref, yb_ref, sxf, sxb, hf, cf, hb, cb):
    i = pl.program_id(0)

    @pl.when(i < NB)
    def _():
        xf = pf_ref[...].reshape(UNROLL * B, D_EMB).astype(jnp.bfloat16)
        xb = pb_ref[...].reshape(UNROLL * B, D_EMB).astype(jnp.bfloat16)
        sxf[i % 2] = _dot_t(xf, wif_ref[...]) + b_ref[:, :G]
        sxb[i % 2] = _dot_t(xb, wib_ref[...]) + b_ref[:, G:]

    @pl.when(i > 0)
    def _():
        _recurrence_block(i, len_ref, sxf, sxb, yf_ref, yb_ref,
                          hf, cf, hb, cb, whf_ref, whb_ref)


def _layer1_body(len_ref, pfa_ref, pfb_ref, pba_ref, pbb_ref,
                 wif_ref, wib_ref, whf_ref, whb_ref, b_ref,
                 yf_ref, yb_ref, sxf, sxb, hf, cf, hb, cb):
    i = pl.program_id(0)

    @pl.when(i < NB)
    def _():
        ya_f = pfa_ref[...].reshape(UNROLL * B, H)
        yb_f = pfb_ref[...].reshape(UNROLL * B, H)
        ya_b = pba_ref[...].reshape(UNROLL * B, H)
        yb_b = pbb_ref[...].reshape(UNROLL * B, H)
        sxf[i % 2] = (_dot_t(ya_f, wif_ref[:, :H]) + _dot_t(yb_f, wif_ref[:, H:])
                      + b_ref[:, :G])
        sxb[i % 2] = (_dot_t(ya_b, wib_ref[:, :H]) + _dot_t(yb_b, wib_ref[:, H:])
                      + b_ref[:, G:])

    @pl.when(i > 0)
    def _():
        _recurrence_block(i, len_ref, sxf, sxb, yf_ref, yb_ref,
                          hf, cf, hb, cb, whf_ref, whb_ref)


def _fwd_map(i):
    j = jnp.minimum(i, NB - 1)
    return (j, 0, 0)


def _bwd_map(i):
    return (NB - 1 - jnp.minimum(i, NB - 1), 0, 0)


def _yf_map(i):
    return (jnp.maximum(i - 1, 0), 0, 0)


def _yb_map(i):
    return (NB - 1 - jnp.maximum(i - 1, 0), 0, 0)


_Y_OUT = [
    jax.ShapeDtypeStruct((T, B, H), jnp.bfloat16),
    jax.ShapeDtypeStruct((T, B, H), jnp.bfloat16),
]

_SCRATCH = [
    pltpu.VMEM((2, UNROLL * B, G), jnp.float32),
    pltpu.VMEM((2, UNROLL * B, G), jnp.float32),
    pltpu.VMEM((B, H), jnp.float32),
    pltpu.VMEM((B, H), jnp.float32),
    pltpu.VMEM((B, H), jnp.float32),
    pltpu.VMEM((B, H), jnp.float32),
]


def _layer0(x0, wif, wib, whf, whb, b, len_col):
    return pl.pallas_call(
        _layer0_body,
        grid=(NB + 1,),
        in_specs=[
            pl.BlockSpec((B, 1), lambda i: (0, 0)),
            pl.BlockSpec((UNROLL, B, D_EMB), _fwd_map),
            pl.BlockSpec((UNROLL, B, D_EMB), _bwd_map),
            pl.BlockSpec((G, D_EMB), lambda i: (0, 0)),
            pl.BlockSpec((G, D_EMB), lambda i: (0, 0)),
            pl.BlockSpec((H, G), lambda i: (0, 0)),
            pl.BlockSpec((H, G), lambda i: (0, 0)),
            pl.BlockSpec((1, 2 * G), lambda i: (0, 0)),
        ],
        out_specs=[
            pl.BlockSpec((UNROLL, B, H), _yf_map),
            pl.BlockSpec((UNROLL, B, H), _yb_map),
        ],
        out_shape=_Y_OUT,
        scratch_shapes=list(_SCRATCH),
        compiler_params=pltpu.CompilerParams(
            dimension_semantics=("arbitrary",)),
    )(len_col, x0, x0, wif, wib, whf, whb, b.reshape(1, 2 * G))


def _layer1(yf0, yb0, wif, wib, whf, whb, b, len_col):
    return pl.pallas_call(
        _layer1_body,
        grid=(NB + 1,),
        in_specs=[
            pl.BlockSpec((B, 1), lambda i: (0, 0)),
            pl.BlockSpec((UNROLL, B, H), _fwd_map),
            pl.BlockSpec((UNROLL, B, H), _fwd_map),
            pl.BlockSpec((UNROLL, B, H), _bwd_map),
            pl.BlockSpec((UNROLL, B, H), _bwd_map),
            pl.BlockSpec((G, 2 * H), lambda i: (0, 0)),
            pl.BlockSpec((G, 2 * H), lambda i: (0, 0)),
            pl.BlockSpec((H, G), lambda i: (0, 0)),
            pl.BlockSpec((H, G), lambda i: (0, 0)),
            pl.BlockSpec((1, 2 * G), lambda i: (0, 0)),
        ],
        out_specs=[
            pl.BlockSpec((UNROLL, B, H), _yf_map),
            pl.BlockSpec((UNROLL, B, H), _yb_map),
        ],
        out_shape=_Y_OUT,
        scratch_shapes=list(_SCRATCH),
        compiler_params=pltpu.CompilerParams(
            dimension_semantics=("arbitrary",)),
    )(len_col, yf0, yb0, yf0, yb0, wif, wib, whf, whb, b.reshape(1, 2 * G))


def _head_body(ya_ref, yb_ref, w_ref, fcb_ref, cls_ref, clsb_ref, o_ref):
    h = (_dot_t(ya_ref[...], w_ref[:, :H]) + _dot_t(yb_ref[...], w_ref[:, H:])
         + fcb_ref[...])
    h = jnp.where(h > 0, h, 0.01 * (jnp.exp(jnp.minimum(h, 0.0)) - 1.0))
    o_ref[...] = _dot_t(h, cls_ref[...]) + clsb_ref[...]


def _head(ya, yb, w, fcb, cls_pad, clsb, bm):
    m = ya.shape[0]
    n = cls_pad.shape[0]
    return pl.pallas_call(
        _head_body,
        grid=(m // bm,),
        in_specs=[
            pl.BlockSpec((bm, H), lambda i: (i, 0)),
            pl.BlockSpec((bm, H), lambda i: (i, 0)),
            pl.BlockSpec((L_OUT, 2 * H), lambda i: (0, 0)),
            pl.BlockSpec((1, L_OUT), lambda i: (0, 0)),
            pl.BlockSpec((n, L_OUT), lambda i: (0, 0)),
            pl.BlockSpec((1, n), lambda i: (0, 0)),
        ],
        out_specs=pl.BlockSpec((bm, n), lambda i: (i, 0)),
        out_shape=jax.ShapeDtypeStruct((m, n), jnp.float32),
    )(ya, yb, w, fcb.reshape(1, L_OUT), cls_pad, clsb.reshape(1, n))


def kernel(inputs, lengths, emb, Wih_l0f, Whh_l0f, bih_l0f, bhh_l0f,
           Wih_l0b, Whh_l0b, bih_l0b, bhh_l0b,
           Wih_l1f, Whh_l1f, bih_l1f, bhh_l1f,
           Wih_l1b, Whh_l1b, bih_l1b, bhh_l1b,
           fc_w, fc_b, cls_w, cls_b):
    f32 = jnp.float32
    bf16 = jnp.bfloat16
    idx = inputs.T.reshape(N).astype(jnp.int32)
    len_col = lengths.astype(f32).reshape(B, 1)

    b0 = jnp.concatenate([bih_l0f + bhh_l0f, bih_l0b + bhh_l0b])
    b1 = jnp.concatenate([bih_l1f + bhh_l1f, bih_l1b + bhh_l1b])
    n_pad = 128
    cls_pad = jnp.zeros((n_pad, L_OUT), f32).at[:TAGS].set(cls_w)
    clsb_pad = jnp.zeros((n_pad,), f32).at[:TAGS].set(cls_b)

    x0 = _sc_gather(emb, idx).reshape(T, B, D_EMB)

    yf0, yb0 = _layer0(x0, Wih_l0f.astype(bf16), Wih_l0b.astype(bf16),
                       Whh_l0f.T.astype(bf16), Whh_l0b.T.astype(bf16),
                       b0, len_col)

    yf1, yb1 = _layer1(yf0, yb0, Wih_l1f.astype(bf16), Wih_l1b.astype(bf16),
                       Whh_l1f.T.astype(bf16), Whh_l1b.T.astype(bf16),
                       b1, len_col)

    out = _head(yf1.reshape(N, H), yb1.reshape(N, H), fc_w.astype(bf16),
                fc_b, cls_pad, clsb_pad, bm=1024)
    return out.reshape(T, B, n_pad).transpose(1, 0, 2)[:, :, :TAGS]


# speedup vs baseline: 15.4881x; 1.0394x over previous
import functools

import jax
import jax.numpy as jnp
from jax.experimental import pallas as pl
from jax.experimental.pallas import tpu as pltpu
from jax.experimental.pallas import tpu_sc as plsc

B, T, VOCAB, D_EMB, H, L_OUT, TAGS = 64, 128, 100000, 256, 512, 256, 50
G = 4 * H
N = T * B
UNROLL = 8
NB = T // UNROLL

_GATHER_WINDOW = 128


def _sc_gather(emb, idx):
    mesh = plsc.VectorSubcoreMesh(core_axis_name="core", subcore_axis_name="subcore")

    @pl.kernel(out_type=jax.ShapeDtypeStruct((N, D_EMB), emb.dtype), mesh=mesh)
    def gather_kernel(emb_hbm, i_hbm, o_hbm):
        def body(i_vmem, o_vmem):
            pltpu.sync_copy(emb_hbm.at[i_vmem.at[0]], o_vmem)

        pltpu.emit_pipeline(
            body,
            grid=(N // _GATHER_WINDOW,),
            in_specs=[pl.BlockSpec((1, _GATHER_WINDOW), index_map=lambda i: (0, i))],
            out_specs=[pl.BlockSpec((_GATHER_WINDOW, D_EMB), index_map=lambda i: (i, 0))],
            core_axis_name=("core", "subcore"),
            dimension_semantics=(pltpu.PARALLEL,),
        )(i_hbm, o_hbm)

    return gather_kernel(emb, idx.reshape(1, N))


def _dot(a, b):
    return jax.lax.dot_general(a, b, (((1,), (0,)), ((), ())),
                               preferred_element_type=jnp.float32)


def _dot_t(a, b):
    return jax.lax.dot_general(a, b, (((1,), (1,)), ((), ())),
                               preferred_element_type=jnp.float32)


def _recurrence_block(i, len_ref, sxf, sxb, yf_ref, yb_ref, hf, cf, hb, cb,
                      whf_ref, whb_ref):
    @pl.when(i == 1)
    def _():
        zero = jnp.zeros((B, H), jnp.float32)
        hf[...] = zero
        cf[...] = zero
        hb[...] = zero
        cb[...] = zero

    lens = len_ref[...]
    buf = (i - 1) % 2

    def gates(g, c):
        gi = jax.nn.sigmoid(g[:, :H])
        gf = jax.nn.sigmoid(g[:, H:2 * H])
        gg = jnp.tanh(g[:, 2 * H:3 * H])
        go = jax.nn.sigmoid(g[:, 3 * H:])
        c_new = gf * c + gi * gg
        h_new = go * jnp.tanh(c_new)
        return h_new, c_new

    t0 = (i - 1) * UNROLL
    for k in range(UNROLL):
        tf = t0 + k
        tb = T - 1 - tf
        g_f = sxf[buf, pl.ds(B * k, B), :] + _dot(hf[...].astype(jnp.bfloat16),
                                                  whf_ref[...])
        g_b = (sxb[buf, pl.ds(B * (UNROLL - 1 - k), B), :]
               + _dot(hb[...].astype(jnp.bfloat16), whb_ref[...]))
        hn_f, cn_f = gates(g_f, cf[...])
        hn_b, cn_b = gates(g_b, cb[...])
        m_f = (lens > jnp.float32(tf)).astype(jnp.float32)
        m_b = (lens > jnp.float32(tb)).astype(jnp.float32)
        yf_ref[k] = (hn_f * m_f).astype(yf_ref.dtype)
        yb_ref[UNROLL - 1 - k] = (hn_b * m_b).astype(yb_ref.dtype)
        hf[...] = m_f * hn_f + (1.0 - m_f) * hf[...]
        cf[...] = m_f * cn_f + (1.0 - m_f) * cf[...]
        hb[...] = m_b * hn_b + (1.0 - m_b) * hb[...]
        cb[...] = m_b * cn_b + (1.0 - m_b) * cb[...]


def _layer0_body(len_ref, pf_ref, pb_ref, wif_ref, wib_ref, whf_ref, whb_ref,
                 b_ref, yf_ref, yb_ref, sxf, sxb, hf, cf, hb, cb):
    i = pl.program_id(0)

    @pl.when(i < NB)
    def _():
        xf = pf_ref[...].reshape(UNROLL * B, D_EMB).astype(jnp.bfloat16)
        xb = pb_ref[...].reshape(UNROLL * B, D_EMB).astype(jnp.bfloat16)
        sxf[i % 2] = (_dot_t(xf, wif_ref[...]) + b_ref[:, :G]).astype(sxf.dtype)
        sxb[i % 2] = (_dot_t(xb, wib_ref[...]) + b_ref[:, G:]).astype(sxb.dtype)

    @pl.when(i > 0)
    def _():
        _recurrence_block(i, len_ref, sxf, sxb, yf_ref, yb_ref,
                          hf, cf, hb, cb, whf_ref, whb_ref)


def _layer1_body(len_ref, pfa_ref, pfb_ref, pba_ref, pbb_ref,
                 wif_ref, wib_ref, whf_ref, whb_ref, b_ref,
                 yf_ref, yb_ref, sxf, sxb, hf, cf, hb, cb):
    i = pl.program_id(0)

    @pl.when(i < NB)
    def _():
        ya_f = pfa_ref[...].reshape(UNROLL * B, H)
        yb_f = pfb_ref[...].reshape(UNROLL * B, H)
        ya_b = pba_ref[...].reshape(UNROLL * B, H)
        yb_b = pbb_ref[...].reshape(UNROLL * B, H)
        sxf[i % 2] = (_dot_t(ya_f, wif_ref[:, :H]) + _dot_t(yb_f, wif_ref[:, H:])
                      + b_ref[:, :G]).astype(sxf.dtype)
        sxb[i % 2] = (_dot_t(ya_b, wib_ref[:, :H]) + _dot_t(yb_b, wib_ref[:, H:])
                      + b_ref[:, G:]).astype(sxb.dtype)

    @pl.when(i > 0)
    def _():
        _recurrence_block(i, len_ref, sxf, sxb, yf_ref, yb_ref,
                          hf, cf, hb, cb, whf_ref, whb_ref)


def _fwd_map(i):
    j = jnp.minimum(i, NB - 1)
    return (j, 0, 0)


def _bwd_map(i):
    return (NB - 1 - jnp.minimum(i, NB - 1), 0, 0)


def _yf_map(i):
    return (jnp.maximum(i - 1, 0), 0, 0)


def _yb_map(i):
    return (NB - 1 - jnp.maximum(i - 1, 0), 0, 0)


_Y_OUT = [
    jax.ShapeDtypeStruct((T, B, H), jnp.bfloat16),
    jax.ShapeDtypeStruct((T, B, H), jnp.bfloat16),
]

_SCRATCH = [
    pltpu.VMEM((2, UNROLL * B, G), jnp.bfloat16),
    pltpu.VMEM((2, UNROLL * B, G), jnp.bfloat16),
    pltpu.VMEM((B, H), jnp.float32),
    pltpu.VMEM((B, H), jnp.float32),
    pltpu.VMEM((B, H), jnp.float32),
    pltpu.VMEM((B, H), jnp.float32),
]


def _layer0(x0, wif, wib, whf, whb, b, len_col):
    return pl.pallas_call(
        _layer0_body,
        grid=(NB + 1,),
        in_specs=[
            pl.BlockSpec((B, 1), lambda i: (0, 0)),
            pl.BlockSpec((UNROLL, B, D_EMB), _fwd_map),
            pl.BlockSpec((UNROLL, B, D_EMB), _bwd_map),
            pl.BlockSpec((G, D_EMB), lambda i: (0, 0)),
            pl.BlockSpec((G, D_EMB), lambda i: (0, 0)),
            pl.BlockSpec((H, G), lambda i: (0, 0)),
            pl.BlockSpec((H, G), lambda i: (0, 0)),
            pl.BlockSpec((1, 2 * G), lambda i: (0, 0)),
        ],
        out_specs=[
            pl.BlockSpec((UNROLL, B, H), _yf_map),
            pl.BlockSpec((UNROLL, B, H), _yb_map),
        ],
        out_shape=_Y_OUT,
        scratch_shapes=list(_SCRATCH),
        compiler_params=pltpu.CompilerParams(
            dimension_semantics=("arbitrary",)),
    )(len_col, x0, x0, wif, wib, whf, whb, b.reshape(1, 2 * G))


def _layer1(yf0, yb0, wif, wib, whf, whb, b, len_col):
    return pl.pallas_call(
        _layer1_body,
        grid=(NB + 1,),
        in_specs=[
            pl.BlockSpec((B, 1), lambda i: (0, 0)),
            pl.BlockSpec((UNROLL, B, H), _fwd_map),
            pl.BlockSpec((UNROLL, B, H), _fwd_map),
            pl.BlockSpec((UNROLL, B, H), _bwd_map),
            pl.BlockSpec((UNROLL, B, H), _bwd_map),
            pl.BlockSpec((G, 2 * H), lambda i: (0, 0)),
            pl.BlockSpec((G, 2 * H), lambda i: (0, 0)),
            pl.BlockSpec((H, G), lambda i: (0, 0)),
            pl.BlockSpec((H, G), lambda i: (0, 0)),
            pl.BlockSpec((1, 2 * G), lambda i: (0, 0)),
        ],
        out_specs=[
            pl.BlockSpec((UNROLL, B, H), _yf_map),
            pl.BlockSpec((UNROLL, B, H), _yb_map),
        ],
        out_shape=_Y_OUT,
        scratch_shapes=list(_SCRATCH),
        compiler_params=pltpu.CompilerParams(
            dimension_semantics=("arbitrary",)),
    )(len_col, yf0, yb0, yf0, yb0, wif, wib, whf, whb, b.reshape(1, 2 * G))


def _head_body(ya_ref, yb_ref, w_ref, fcb_ref, cls_ref, clsb_ref, o_ref):
    h = (_dot_t(ya_ref[...], w_ref[:, :H]) + _dot_t(yb_ref[...], w_ref[:, H:])
         + fcb_ref[...])
    h = jnp.where(h > 0, h, 0.01 * (jnp.exp(jnp.minimum(h, 0.0)) - 1.0))
    o_ref[...] = _dot_t(h, cls_ref[...]) + clsb_ref[...]


def _head(ya, yb, w, fcb, cls_pad, clsb, bm):
    m = ya.shape[0]
    n = cls_pad.shape[0]
    return pl.pallas_call(
        _head_body,
        grid=(m // bm,),
        in_specs=[
            pl.BlockSpec((bm, H), lambda i: (i, 0)),
            pl.BlockSpec((bm, H), lambda i: (i, 0)),
            pl.BlockSpec((L_OUT, 2 * H), lambda i: (0, 0)),
            pl.BlockSpec((1, L_OUT), lambda i: (0, 0)),
            pl.BlockSpec((n, L_OUT), lambda i: (0, 0)),
            pl.BlockSpec((1, n), lambda i: (0, 0)),
        ],
        out_specs=pl.BlockSpec((bm, n), lambda i: (i, 0)),
        out_shape=jax.ShapeDtypeStruct((m, n), jnp.float32),
    )(ya, yb, w, fcb.reshape(1, L_OUT), cls_pad, clsb.reshape(1, n))


def kernel(inputs, lengths, emb, Wih_l0f, Whh_l0f, bih_l0f, bhh_l0f,
           Wih_l0b, Whh_l0b, bih_l0b, bhh_l0b,
           Wih_l1f, Whh_l1f, bih_l1f, bhh_l1f,
           Wih_l1b, Whh_l1b, bih_l1b, bhh_l1b,
           fc_w, fc_b, cls_w, cls_b):
    f32 = jnp.float32
    bf16 = jnp.bfloat16
    idx = inputs.T.reshape(N).astype(jnp.int32)
    len_col = lengths.astype(f32).reshape(B, 1)

    b0 = jnp.concatenate([bih_l0f + bhh_l0f, bih_l0b + bhh_l0b])
    b1 = jnp.concatenate([bih_l1f + bhh_l1f, bih_l1b + bhh_l1b])
    n_pad = 128
    cls_pad = jnp.zeros((n_pad, L_OUT), f32).at[:TAGS].set(cls_w)
    clsb_pad = jnp.zeros((n_pad,), f32).at[:TAGS].set(cls_b)

    x0 = _sc_gather(emb, idx).reshape(T, B, D_EMB)

    yf0, yb0 = _layer0(x0, Wih_l0f.astype(bf16), Wih_l0b.astype(bf16),
                       Whh_l0f.T.astype(bf16), Whh_l0b.T.astype(bf16),
                       b0, len_col)

    yf1, yb1 = _layer1(yf0, yb0, Wih_l1f.astype(bf16), Wih_l1b.astype(bf16),
                       Whh_l1f.T.astype(bf16), Whh_l1b.T.astype(bf16),
                       b1, len_col)

    out = _head(yf1.reshape(N, H), yb1.reshape(N, H), fc_w.astype(bf16),
                fc_b, cls_pad, clsb_pad, bm=1024)
    return out.reshape(T, B, n_pad).transpose(1, 0, 2)[:, :, :TAGS]


# speedup vs baseline: 15.7583x; 1.0174x over previous
import functools

import jax
import jax.numpy as jnp
from jax.experimental import pallas as pl
from jax.experimental.pallas import tpu as pltpu
from jax.experimental.pallas import tpu_sc as plsc

B, T, VOCAB, D_EMB, H, L_OUT, TAGS = 64, 128, 100000, 256, 512, 256, 50
G = 4 * H
N = T * B
UNROLL = 16
NB = T // UNROLL

_GATHER_WINDOW = 128


def _sc_gather(emb, idx):
    mesh = plsc.VectorSubcoreMesh(core_axis_name="core", subcore_axis_name="subcore")

    @pl.kernel(out_type=jax.ShapeDtypeStruct((N, D_EMB), emb.dtype), mesh=mesh)
    def gather_kernel(emb_hbm, i_hbm, o_hbm):
        def body(i_vmem, o_vmem):
            pltpu.sync_copy(emb_hbm.at[i_vmem.at[0]], o_vmem)

        pltpu.emit_pipeline(
            body,
            grid=(N // _GATHER_WINDOW,),
            in_specs=[pl.BlockSpec((1, _GATHER_WINDOW), index_map=lambda i: (0, i))],
            out_specs=[pl.BlockSpec((_GATHER_WINDOW, D_EMB), index_map=lambda i: (i, 0))],
            core_axis_name=("core", "subcore"),
            dimension_semantics=(pltpu.PARALLEL,),
        )(i_hbm, o_hbm)

    return gather_kernel(emb, idx.reshape(1, N))


def _dot(a, b):
    return jax.lax.dot_general(a, b, (((1,), (0,)), ((), ())),
                               preferred_element_type=jnp.float32)


def _dot_t(a, b):
    return jax.lax.dot_general(a, b, (((1,), (1,)), ((), ())),
                               preferred_element_type=jnp.float32)


def _recurrence_block(i, len_ref, sxf, sxb, yf_ref, yb_ref, hf, cf, hb, cb,
                      whf_ref, whb_ref):
    @pl.when(i == 1)
    def _():
        zero = jnp.zeros((B, H), jnp.float32)
        hf[...] = zero
        cf[...] = zero
        hb[...] = zero
        cb[...] = zero

    lens = len_ref[...]
    buf = (i - 1) % 2

    def gates(g, c):
        gi = jax.nn.sigmoid(g[:, :H])
        gf = jax.nn.sigmoid(g[:, H:2 * H])
        gg = jnp.tanh(g[:, 2 * H:3 * H])
        go = jax.nn.sigmoid(g[:, 3 * H:])
        c_new = gf * c + gi * gg
        h_new = go * jnp.tanh(c_new)
        return h_new, c_new

    t0 = (i - 1) * UNROLL
    for k in range(UNROLL):
        tf = t0 + k
        tb = T - 1 - tf
        g_f = sxf[buf, pl.ds(B * k, B), :] + _dot(hf[...].astype(jnp.bfloat16),
                                                  whf_ref[...])
        g_b = (sxb[buf, pl.ds(B * (UNROLL - 1 - k), B), :]
               + _dot(hb[...].astype(jnp.bfloat16), whb_ref[...]))
        hn_f, cn_f = gates(g_f, cf[...])
        hn_b, cn_b = gates(g_b, cb[...])
        m_f = (lens > jnp.float32(tf)).astype(jnp.float32)
        m_b = (lens > jnp.float32(tb)).astype(jnp.float32)
        yf_ref[k] = (hn_f * m_f).astype(yf_ref.dtype)
        yb_ref[UNROLL - 1 - k] = (hn_b * m_b).astype(yb_ref.dtype)
        hf[...] = m_f * hn_f + (1.0 - m_f) * hf[...]
        cf[...] = m_f * cn_f + (1.0 - m_f) * cf[...]
        hb[...] = m_b * hn_b + (1.0 - m_b) * hb[...]
        cb[...] = m_b * cn_b + (1.0 - m_b) * cb[...]


def _layer0_body(len_ref, pf_ref, pb_ref, wif_ref, wib_ref, whf_ref, whb_ref,
                 b_ref, yf_ref, yb_ref, sxf, sxb, hf, cf, hb, cb):
    i = pl.program_id(0)

    @pl.when(i < NB)
    def _():
        xf = pf_ref[...].reshape(UNROLL * B, D_EMB).astype(jnp.bfloat16)
        xb = pb_ref[...].reshape(UNROLL * B, D_EMB).astype(jnp.bfloat16)
        sxf[i % 2] = (_dot_t(xf, wif_ref[...]) + b_ref[:, :G]).astype(sxf.dtype)
        sxb[i % 2] = (_dot_t(xb, wib_ref[...]) + b_ref[:, G:]).astype(sxb.dtype)

    @pl.when(i > 0)
    def _():
        _recurrence_block(i, len_ref, sxf, sxb, yf_ref, yb_ref,
                          hf, cf, hb, cb, whf_ref, whb_ref)


def _layer1_body(len_ref, pfa_ref, pfb_ref, pba_ref, pbb_ref,
                 wif_ref, wib_ref, whf_ref, whb_ref, b_ref,
                 yf_ref, yb_ref, sxf, sxb, hf, cf, hb, cb):
    i = pl.program_id(0)

    @pl.when(i < NB)
    def _():
        ya_f = pfa_ref[...].reshape(UNROLL * B, H)
        yb_f = pfb_ref[...].reshape(UNROLL * B, H)
        ya_b = pba_ref[...].reshape(UNROLL * B, H)
        yb_b = pbb_ref[...].reshape(UNROLL * B, H)
        sxf[i % 2] = (_dot_t(ya_f, wif_ref[:, :H]) + _dot_t(yb_f, wif_ref[:, H:])
                      + b_ref[:, :G]).astype(sxf.dtype)
        sxb[i % 2] = (_dot_t(ya_b, wib_ref[:, :H]) + _dot_t(yb_b, wib_ref[:, H:])
                      + b_ref[:, G:]).astype(sxb.dtype)

    @pl.when(i > 0)
    def _():
        _recurrence_block(i, len_ref, sxf, sxb, yf_ref, yb_ref,
                          hf, cf, hb, cb, whf_ref, whb_ref)


def _fwd_map(i):
    j = jnp.minimum(i, NB - 1)
    return (j, 0, 0)


def _bwd_map(i):
    return (NB - 1 - jnp.minimum(i, NB - 1), 0, 0)


def _yf_map(i):
    return (jnp.maximum(i - 1, 0), 0, 0)


def _yb_map(i):
    return (NB - 1 - jnp.maximum(i - 1, 0), 0, 0)


_Y_OUT = [
    jax.ShapeDtypeStruct((T, B, H), jnp.bfloat16),
    jax.ShapeDtypeStruct((T, B, H), jnp.bfloat16),
]

_SCRATCH = [
    pltpu.VMEM((2, UNROLL * B, G), jnp.bfloat16),
    pltpu.VMEM((2, UNROLL * B, G), jnp.bfloat16),
    pltpu.VMEM((B, H), jnp.float32),
    pltpu.VMEM((B, H), jnp.float32),
    pltpu.VMEM((B, H), jnp.float32),
    pltpu.VMEM((B, H), jnp.float32),
]


def _layer0(x0, wif, wib, whf, whb, b, len_col):
    return pl.pallas_call(
        _layer0_body,
        grid=(NB + 1,),
        in_specs=[
            pl.BlockSpec((B, 1), lambda i: (0, 0)),
            pl.BlockSpec((UNROLL, B, D_EMB), _fwd_map),
            pl.BlockSpec((UNROLL, B, D_EMB), _bwd_map),
            pl.BlockSpec((G, D_EMB), lambda i: (0, 0)),
            pl.BlockSpec((G, D_EMB), lambda i: (0, 0)),
            pl.BlockSpec((H, G), lambda i: (0, 0)),
            pl.BlockSpec((H, G), lambda i: (0, 0)),
            pl.BlockSpec((1, 2 * G), lambda i: (0, 0)),
        ],
        out_specs=[
            pl.BlockSpec((UNROLL, B, H), _yf_map),
            pl.BlockSpec((UNROLL, B, H), _yb_map),
        ],
        out_shape=_Y_OUT,
        scratch_shapes=list(_SCRATCH),
        compiler_params=pltpu.CompilerParams(
            dimension_semantics=("arbitrary",)),
    )(len_col, x0, x0, wif, wib, whf, whb, b.reshape(1, 2 * G))


def _layer1(yf0, yb0, wif, wib, whf, whb, b, len_col):
    return pl.pallas_call(
        _layer1_body,
        grid=(NB + 1,),
        in_specs=[
            pl.BlockSpec((B, 1), lambda i: (0, 0)),
            pl.BlockSpec((UNROLL, B, H), _fwd_map),
            pl.BlockSpec((UNROLL, B, H), _fwd_map),
            pl.BlockSpec((UNROLL, B, H), _bwd_map),
            pl.BlockSpec((UNROLL, B, H), _bwd_map),
            pl.BlockSpec((G, 2 * H), lambda i: (0, 0)),
            pl.BlockSpec((G, 2 * H), lambda i: (0, 0)),
            pl.BlockSpec((H, G), lambda i: (0, 0)),
            pl.BlockSpec((H, G), lambda i: (0, 0)),
            pl.BlockSpec((1, 2 * G), lambda i: (0, 0)),
        ],
        out_specs=[
            pl.BlockSpec((UNROLL, B, H), _yf_map),
            pl.BlockSpec((UNROLL, B, H), _yb_map),
        ],
        out_shape=_Y_OUT,
        scratch_shapes=list(_SCRATCH),
        compiler_params=pltpu.CompilerParams(
            dimension_semantics=("arbitrary",)),
    )(len_col, yf0, yb0, yf0, yb0, wif, wib, whf, whb, b.reshape(1, 2 * G))


def _head_body(ya_ref, yb_ref, w_ref, fcb_ref, cls_ref, clsb_ref, o_ref):
    h = (_dot_t(ya_ref[...], w_ref[:, :H]) + _dot_t(yb_ref[...], w_ref[:, H:])
         + fcb_ref[...])
    h = jnp.where(h > 0, h, 0.01 * (jnp.exp(jnp.minimum(h, 0.0)) - 1.0))
    o_ref[...] = _dot_t(h, cls_ref[...]) + clsb_ref[...]


def _head(ya, yb, w, fcb, cls_pad, clsb, bm):
    m = ya.shape[0]
    n = cls_pad.shape[0]
    return pl.pallas_call(
        _head_body,
        grid=(m // bm,),
        in_specs=[
            pl.BlockSpec((bm, H), lambda i: (i, 0)),
            pl.BlockSpec((bm, H), lambda i: (i, 0)),
            pl.BlockSpec((L_OUT, 2 * H), lambda i: (0, 0)),
            pl.BlockSpec((1, L_OUT), lambda i: (0, 0)),
            pl.BlockSpec((n, L_OUT), lambda i: (0, 0)),
            pl.BlockSpec((1, n), lambda i: (0, 0)),
        ],
        out_specs=pl.BlockSpec((bm, n), lambda i: (i, 0)),
        out_shape=jax.ShapeDtypeStruct((m, n), jnp.float32),
    )(ya, yb, w, fcb.reshape(1, L_OUT), cls_pad, clsb.reshape(1, n))


def kernel(inputs, lengths, emb, Wih_l0f, Whh_l0f, bih_l0f, bhh_l0f,
           Wih_l0b, Whh_l0b, bih_l0b, bhh_l0b,
           Wih_l1f, Whh_l1f, bih_l1f, bhh_l1f,
           Wih_l1b, Whh_l1b, bih_l1b, bhh_l1b,
           fc_w, fc_b, cls_w, cls_b):
    f32 = jnp.float32
    bf16 = jnp.bfloat16
    idx = inputs.T.reshape(N).astype(jnp.int32)
    len_col = lengths.astype(f32).reshape(B, 1)

    b0 = jnp.concatenate([bih_l0f + bhh_l0f, bih_l0b + bhh_l0b])
    b1 = jnp.concatenate([bih_l1f + bhh_l1f, bih_l1b + bhh_l1b])
    n_pad = 128
    cls_pad = jnp.zeros((n_pad, L_OUT), f32).at[:TAGS].set(cls_w)
    clsb_pad = jnp.zeros((n_pad,), f32).at[:TAGS].set(cls_b)

    x0 = _sc_gather(emb, idx).reshape(T, B, D_EMB)

    yf0, yb0 = _layer0(x0, Wih_l0f.astype(bf16), Wih_l0b.astype(bf16),
                       Whh_l0f.T.astype(bf16), Whh_l0b.T.astype(bf16),
                       b0, len_col)

    yf1, yb1 = _layer1(yf0, yb0, Wih_l1f.astype(bf16), Wih_l1b.astype(bf16),
                       Whh_l1f.T.astype(bf16), Whh_l1b.T.astype(bf16),
                       b1, len_col)

    out = _head(yf1.reshape(N, H), yb1.reshape(N, H), fc_w.astype(bf16),
                fc_b, cls_pad, clsb_pad, bm=1024)
    return out.reshape(T, B, n_pad).transpose(1, 0, 2)[:, :, :TAGS]
